# baseline (speedup 1.0000x reference)
import functools

import jax
import jax.numpy as jnp
from jax import lax
from jax.experimental import pallas as pl
from jax.experimental.pallas import tpu as pltpu
from jax.experimental.pallas import tpu_sc as plsc

N_USERS = 100000
D = 128
N_NODES = 10000
DEG = 32

NW = 32
CHUNK = 128

NSLICE = 5
NS = N_NODES // NSLICE
_RAW = NS * DEG + NS
_TOT = ((_RAW + NW * CHUNK - 1) // (NW * CHUNK)) * (NW * CHUNK)
_PER_W = _TOT // NW
_N_CHUNK = _PER_W // CHUNK

BN = 200
GRID = NS // BN


def _sc_gather(idx, table):
    mesh = plsc.VectorSubcoreMesh(core_axis_name="c", subcore_axis_name="s")

    @functools.partial(
        pl.kernel,
        out_type=jax.ShapeDtypeStruct((_TOT, D), jnp.float32),
        mesh=mesh,
        scratch_types=[
            pltpu.VMEM((CHUNK,), jnp.int32),
            pltpu.VMEM((CHUNK, D), jnp.float32),
            pltpu.SemaphoreType.DMA,
        ],
    )
    def gather_kernel(idx_hbm, table_hbm, out_hbm, idx_v, rows_v, sem):
        nc = 2
        wid = lax.axis_index("s") * nc + lax.axis_index("c")
        base = wid * _PER_W

        def body(c, _):
            off = base + c * CHUNK
            pltpu.sync_copy(idx_hbm.at[pl.ds(off, CHUNK)], idx_v)
            pltpu.async_copy(table_hbm.at[idx_v], rows_v, sem).wait()
            pltpu.sync_copy(rows_v, out_hbm.at[pl.ds(off, CHUNK)])
            return ()

        lax.fori_loop(0, _N_CHUNK, body, ())

    return gather_kernel(idx, table)


def _tc_body(eu_ref, u_ref, w1a_ref, w1b_ref, b1_ref, w2_ref, b2_ref,
             w3_ref, out_ref):
    eu = eu_ref[...]
    u = u_ref[...]
    u_part = jnp.dot(u, w1b_ref[...],
                     preferred_element_type=jnp.float32) + b1_ref[...]
    h = jnp.dot(eu, w1a_ref[...], preferred_element_type=jnp.float32)
    h = h.reshape(BN, DEG, D) + u_part[:, None, :]
    h = jnp.maximum(h, 0.0).reshape(BN * DEG, D)
    h = jnp.dot(h, w2_ref[...], preferred_element_type=jnp.float32)
    h = jnp.maximum(h + b2_ref[...], 0.0)
    logits = jnp.sum(h * w3_ref[...], axis=1).reshape(BN, DEG)
    logits = logits - jnp.max(logits, axis=1, keepdims=True)
    e = jnp.exp(logits)
    att = e / jnp.sum(e, axis=1, keepdims=True)
    w = att[:, :, None] * eu.reshape(BN, DEG, D)
    out_ref[...] = jnp.sum(w, axis=1)


def _tc_call(rows, w1a, w1b, b1r, W2, b2r, w3r):
    grid_spec = pl.GridSpec(
        grid=(GRID,),
        in_specs=[
            pl.BlockSpec((BN * DEG, D), lambda i: (i, 0)),
            pl.BlockSpec((BN, D), lambda i: (NS * DEG // BN + i, 0)),
            pl.BlockSpec((D, D), lambda i: (0, 0)),
            pl.BlockSpec((D, D), lambda i: (0, 0)),
            pl.BlockSpec((1, D), lambda i: (0, 0)),
            pl.BlockSpec((D, D), lambda i: (0, 0)),
            pl.BlockSpec((1, D), lambda i: (0, 0)),
            pl.BlockSpec((1, D), lambda i: (0, 0)),
        ],
        out_specs=pl.BlockSpec((BN, D), lambda i: (i, 0)),
    )
    return pl.pallas_call(
        _tc_body,
        grid_spec=grid_spec,
        out_shape=jax.ShapeDtypeStruct((NS, D), jnp.float32),
        compiler_params=pltpu.CompilerParams(
            dimension_semantics=("arbitrary",),
        ),
    )(rows, rows, w1a, w1b, b1r, W2, b2r, w3r)


def kernel(nodes, to_neighs, u2e, W1, b1, W2, b2, W3, b3):
    w1a = W1[:D]
    w1b = W1[D:]
    b1r = b1.reshape(1, D)
    b2r = b2.reshape(1, D)
    w3r = W3.reshape(1, D)

    pad = jnp.zeros((_TOT - _RAW,), jnp.int32)
    outs = []
    for s in range(NSLICE):
        idx = jnp.concatenate([
            lax.dynamic_slice_in_dim(to_neighs, s * NS, NS, 0).reshape(-1),
            lax.dynamic_slice_in_dim(nodes, s * NS, NS, 0),
            pad,
        ])
        rows = _sc_gather(idx, u2e)
        outs.append(_tc_call(rows, w1a, w1b, b1r, W2, b2r, w3r))
    return jnp.concatenate(outs, axis=0)


# speedup vs baseline: 2.2332x; 2.2332x over previous
import functools

import jax
import jax.numpy as jnp
from jax import lax
from jax.experimental import pallas as pl
from jax.experimental.pallas import tpu as pltpu
from jax.experimental.pallas import tpu_sc as plsc

N_USERS = 100000
D = 128
N_NODES = 10000
DEG = 32

NW = 32
CHUNK = 128

NSLICE = 1
NS = N_NODES // NSLICE
_RAW = NS * DEG + NS
_TOT = ((_RAW + NW * CHUNK - 1) // (NW * CHUNK)) * (NW * CHUNK)
_PER_W = _TOT // NW
_N_CHUNK = _PER_W // CHUNK

BN = 200
GRID = NS // BN


def _sc_gather(idx, table):
    mesh = plsc.VectorSubcoreMesh(core_axis_name="c", subcore_axis_name="s")

    @functools.partial(
        pl.kernel,
        out_type=jax.ShapeDtypeStruct((_TOT, D), jnp.float32),
        mesh=mesh,
        scratch_types=[
            pltpu.VMEM((_PER_W,), jnp.int32),
            pltpu.VMEM((CHUNK, D), jnp.float32),
            pltpu.SemaphoreType.DMA,
        ],
    )
    def gather_kernel(idx_hbm, table_hbm, out_hbm, idx_all, rows_v, sem):
        nc = 2
        wid = lax.axis_index("s") * nc + lax.axis_index("c")
        base = wid * _PER_W
        pltpu.sync_copy(idx_hbm.at[pl.ds(base, _PER_W)], idx_all)

        def body(c, _):
            off = base + c * CHUNK
            pltpu.async_copy(
                table_hbm.at[idx_all.at[pl.ds(c * CHUNK, CHUNK)]],
                rows_v, sem).wait()
            pltpu.sync_copy(rows_v, out_hbm.at[pl.ds(off, CHUNK)])
            return ()

        lax.fori_loop(0, _N_CHUNK, body, ())

    return gather_kernel(idx, table)


def _tc_body(eu_ref, u_ref, w1a_ref, w1b_ref, b1_ref, w2_ref, b2_ref,
             w3_ref, out_ref):
    eu = eu_ref[...]
    u = u_ref[...]
    u_part = jnp.dot(u, w1b_ref[...],
                     preferred_element_type=jnp.float32) + b1_ref[...]
    h = jnp.dot(eu, w1a_ref[...], preferred_element_type=jnp.float32)
    h = h.reshape(BN, DEG, D) + u_part[:, None, :]
    h = jnp.maximum(h, 0.0).reshape(BN * DEG, D)
    h = jnp.dot(h, w2_ref[...], preferred_element_type=jnp.float32)
    h = jnp.maximum(h + b2_ref[...], 0.0)
    logits = jnp.sum(h * w3_ref[...], axis=1).reshape(BN, DEG)
    logits = logits - jnp.max(logits, axis=1, keepdims=True)
    e = jnp.exp(logits)
    att = e / jnp.sum(e, axis=1, keepdims=True)
    w = att[:, :, None] * eu.reshape(BN, DEG, D)
    out_ref[...] = jnp.sum(w, axis=1)


def _tc_call(rows, w1a, w1b, b1r, W2, b2r, w3r):
    grid_spec = pl.GridSpec(
        grid=(GRID,),
        in_specs=[
            pl.BlockSpec((BN * DEG, D), lambda i: (i, 0)),
            pl.BlockSpec((BN, D), lambda i: (NS * DEG // BN + i, 0)),
            pl.BlockSpec((D, D), lambda i: (0, 0)),
            pl.BlockSpec((D, D), lambda i: (0, 0)),
            pl.BlockSpec((1, D), lambda i: (0, 0)),
            pl.BlockSpec((D, D), lambda i: (0, 0)),
            pl.BlockSpec((1, D), lambda i: (0, 0)),
            pl.BlockSpec((1, D), lambda i: (0, 0)),
        ],
        out_specs=pl.BlockSpec((BN, D), lambda i: (i, 0)),
    )
    return pl.pallas_call(
        _tc_body,
        grid_spec=grid_spec,
        out_shape=jax.ShapeDtypeStruct((NS, D), jnp.float32),
        compiler_params=pltpu.CompilerParams(
            dimension_semantics=("arbitrary",),
        ),
    )(rows, rows, w1a, w1b, b1r, W2, b2r, w3r)


def kernel(nodes, to_neighs, u2e, W1, b1, W2, b2, W3, b3):
    w1a = W1[:D]
    w1b = W1[D:]
    b1r = b1.reshape(1, D)
    b2r = b2.reshape(1, D)
    w3r = W3.reshape(1, D)

    pad = jnp.zeros((_TOT - _RAW,), jnp.int32)
    outs = []
    for s in range(NSLICE):
        idx = jnp.concatenate([
            lax.dynamic_slice_in_dim(to_neighs, s * NS, NS, 0).reshape(-1),
            lax.dynamic_slice_in_dim(nodes, s * NS, NS, 0),
            pad,
        ])
        rows = _sc_gather(idx, u2e)
        outs.append(_tc_call(rows, w1a, w1b, b1r, W2, b2r, w3r))
    return jnp.concatenate(outs, axis=0)
